# baseline (device time: 9335 ns/iter reference)
import jax
import jax.numpy as jnp
from jax import lax
from jax.experimental import pallas as pl
from jax.experimental.pallas import tpu as pltpu

N_DEV = 4


def kernel(x):
    m, n = x.shape

    def body(x_ref, out_ref, send_ref, recv_ref, send_sems, recv_sems):
        my = lax.axis_index("i")

        barrier_sem = pltpu.get_barrier_semaphore()
        for k in range(1, N_DEV):
            pl.semaphore_signal(
                barrier_sem,
                inc=1,
                device_id=((my + k) % N_DEV,),
                device_id_type=pl.DeviceIdType.MESH,
            )
        send_ref[...] = x_ref[...].astype(jnp.bfloat16)
        pl.semaphore_wait(barrier_sem, N_DEV - 1)

        rdmas = {}
        for k in (2, 1, 3):
            rdma = pltpu.make_async_remote_copy(
                src_ref=send_ref,
                dst_ref=recv_ref.at[k - 1],
                send_sem=send_sems.at[k - 1],
                recv_sem=recv_sems.at[k - 1],
                device_id=((my + k) % N_DEV,),
                device_id_type=pl.DeviceIdType.MESH,
            )
            rdma.start()
            rdmas[k] = rdma

        rdmas[1].wait_recv()
        rdmas[3].wait_recv()
        acc = send_ref[...] + recv_ref[0] + recv_ref[2]
        rdmas[2].wait_recv()
        out_ref[...] = acc + recv_ref[1]

        for rdma in rdmas.values():
            rdma.wait_send()

    return pl.pallas_call(
        body,
        out_shape=jax.ShapeDtypeStruct((m, n), jnp.bfloat16),
        in_specs=[pl.BlockSpec(memory_space=pltpu.VMEM)],
        out_specs=pl.BlockSpec(memory_space=pltpu.VMEM),
        scratch_shapes=[
            pltpu.VMEM((m, n), jnp.bfloat16),
            pltpu.VMEM((N_DEV - 1, m, n), jnp.bfloat16),
            pltpu.SemaphoreType.DMA((N_DEV - 1,)),
            pltpu.SemaphoreType.DMA((N_DEV - 1,)),
        ],
        compiler_params=pltpu.CompilerParams(collective_id=0),
    )(x)


# device time: 9147 ns/iter; 1.0206x vs baseline; 1.0206x over previous
import jax
import jax.numpy as jnp
from jax import lax
from jax.experimental import pallas as pl
from jax.experimental.pallas import tpu as pltpu

N_DEV = 4


def kernel(x):
    m, n = x.shape
    h = m // 2

    def body(x_ref, out_ref, send_ref, s2_ref, r1_ref, r2_ref,
             send_sems, recv_sems):
        my = lax.axis_index("i")
        s = 1 - 2 * (my % 2)
        pa = (my + s) % N_DEV
        pb = (my - s + N_DEV) % N_DEV

        barrier_sem = pltpu.get_barrier_semaphore()
        for nbr in (pa, pb):
            pl.semaphore_signal(
                barrier_sem,
                inc=1,
                device_id=(nbr,),
                device_id_type=pl.DeviceIdType.MESH,
            )
        send_ref[...] = x_ref[...].astype(jnp.bfloat16)
        pl.semaphore_wait(barrier_sem, 2)

        s1a = pltpu.make_async_remote_copy(
            src_ref=send_ref.at[pl.ds(0, h)],
            dst_ref=r1_ref.at[0],
            send_sem=send_sems.at[0],
            recv_sem=recv_sems.at[0],
            device_id=(pa,),
            device_id_type=pl.DeviceIdType.MESH,
        )
        s1b = pltpu.make_async_remote_copy(
            src_ref=send_ref.at[pl.ds(h, h)],
            dst_ref=r1_ref.at[1],
            send_sem=send_sems.at[1],
            recv_sem=recv_sems.at[1],
            device_id=(pb,),
            device_id_type=pl.DeviceIdType.MESH,
        )
        s1a.start()
        s1b.start()

        s1b.wait_recv()
        s2_ref[1] = send_ref[pl.ds(h, h)] + r1_ref[1]
        s2a = pltpu.make_async_remote_copy(
            src_ref=s2_ref.at[1],
            dst_ref=r2_ref.at[1],
            send_sem=send_sems.at[2],
            recv_sem=recv_sems.at[2],
            device_id=(pa,),
            device_id_type=pl.DeviceIdType.MESH,
        )
        s2a.start()

        s1a.wait_recv()
        s2_ref[0] = send_ref[pl.ds(0, h)] + r1_ref[0]
        s2b = pltpu.make_async_remote_copy(
            src_ref=s2_ref.at[0],
            dst_ref=r2_ref.at[0],
            send_sem=send_sems.at[3],
            recv_sem=recv_sems.at[3],
            device_id=(pb,),
            device_id_type=pl.DeviceIdType.MESH,
        )
        s2b.start()

        s2b.wait_recv()
        out_ref[pl.ds(0, h), :] = s2_ref[0] + r2_ref[0]
        s2a.wait_recv()
        out_ref[pl.ds(h, h), :] = s2_ref[1] + r2_ref[1]

        for rdma in (s1a, s1b, s2a, s2b):
            rdma.wait_send()

    return pl.pallas_call(
        body,
        out_shape=jax.ShapeDtypeStruct((m, n), jnp.bfloat16),
        in_specs=[pl.BlockSpec(memory_space=pltpu.VMEM)],
        out_specs=pl.BlockSpec(memory_space=pltpu.VMEM),
        scratch_shapes=[
            pltpu.VMEM((m, n), jnp.bfloat16),
            pltpu.VMEM((2, h, n), jnp.bfloat16),
            pltpu.VMEM((2, h, n), jnp.bfloat16),
            pltpu.VMEM((2, h, n), jnp.bfloat16),
            pltpu.SemaphoreType.DMA((4,)),
            pltpu.SemaphoreType.DMA((4,)),
        ],
        compiler_params=pltpu.CompilerParams(collective_id=0),
    )(x)
